# baseline (speedup 1.0000x reference)
import jax
import jax.numpy as jnp
from jax import lax
from jax.experimental import pallas as pl
from jax.experimental.pallas import tpu as pltpu
from jax.experimental.pallas import tpu_sc as plsc

BATCH = 16384
EMBED_DIM = 32
NUM_CORES = 2
NUM_SUBCORES = 16
NUM_WORKERS = NUM_CORES * NUM_SUBCORES
USER_COUNT = 1000000
ITEM_COUNT = 100000
SLAB = 512
U_SLABS = USER_COUNT // SLAB
I_SLABS = ITEM_COUNT // SLAB
U_TAIL = U_SLABS * SLAB
I_TAIL = I_SLABS * SLAB
U_SPW = -(-U_SLABS // NUM_WORKERS)
I_SPW = -(-I_SLABS // NUM_WORKERS)
OUT_ROWS = BATCH + NUM_WORKERS
BLK = 2048
_I16 = lambda: lax.iota(jnp.int32, 16)


def _sweep_table(tab, out, idx_v, hits, sub, slabs, ostage, bst,
                 ssem, osem, wid, n_slabs, spw):
    lo = wid * spw
    hi = jnp.minimum(lo + spw, n_slabs)
    dump = BATCH + wid

    def scan(k, cnt):
        v = idx_v[pl.ds(k * 16, 16)]
        sg = lax.shift_right_logical(v, 9)
        m = (sg >= lo) & (sg < hi)
        packed = (((sg - lo) << 23) | ((v & (SLAB - 1)) << 14)
                  | (k * 16 + _I16()))
        pos = cnt + plsc.cumsum(m.astype(jnp.int32)) - 1
        plsc.store_scatter(hits, [pos], packed, mask=m)
        return cnt + plsc.all_reduce_population_count(m)[0]

    cnt = lax.fori_loop(0, BATCH // 16, scan, jnp.int32(0))
    plsc.store_scatter(hits, [cnt + _I16()],
                       jnp.full((16,), 63 << 23, jnp.int32))
    nch = lax.shift_right_logical(cnt + 15, 4)

    def process(s_local, buf, ocount):
        def rescan(t, scnt):
            hv = hits[pl.ds(t * 16, 16)]
            m = lax.shift_right_logical(hv, 23) == s_local
            pos = scnt + plsc.cumsum(m.astype(jnp.int32)) - 1
            plsc.store_scatter(sub, [pos], hv, mask=m)
            return scnt + plsc.all_reduce_population_count(m)[0]

        scnt = lax.fori_loop(0, nch, rescan, jnp.int32(0))

        def extract(e, oc):
            slot = oc & 1
            og = ostage.at[slot]

            @pl.when(oc >= 2)
            def _():
                pltpu.make_async_copy(og, out.at[pl.ds(0, 32)], osem).wait()

            ball = jnp.zeros((32,), jnp.int32)
            for g in range(2):
                hv = sub[pl.ds(e * 32 + g * 16, 16)]
                col = lax.shift_right_logical(hv, 14) & (SLAB - 1)
                valid = (e * 32 + g * 16 + _I16()) < scnt
                b = jnp.where(valid, hv & (BATCH - 1), dump)
                ball = ball.at[pl.ds(g * 16, 16)].set(b) if False else ball
                bst.at[slot][pl.ds(g * 16, 16)] = b
                for f in range(EMBED_DIM):
                    vals = plsc.load_gather(
                        buf, [jnp.full((16,), f, jnp.int32), col])
                    plsc.store_scatter(
                        og, [g * 16 + _I16(), jnp.full((16,), f, jnp.int32)],
                        vals)
            pltpu.async_copy(og, out.at[bst.at[slot]], osem)
            return oc + 1

        nech = lax.shift_right_logical(scnt + 31, 5)
        return lax.fori_loop(0, nech, extract, ocount)

    def fire(sg, buf):
        @pl.when(sg < hi)
        def _():
            off = pl.multiple_of(sg * SLAB, SLAB)
            pltpu.async_copy(tab.at[:, pl.ds(off, SLAB)], buf, ssem)

    def drain(sg, buf):
        @pl.when(sg < hi)
        def _():
            pltpu.make_async_copy(tab.at[:, pl.ds(0, SLAB)], buf, ssem).wait()

    fire(lo, slabs.at[0])
    fire(lo + 1, slabs.at[1])

    def step(s2, ocount):
        se = lo + 2 * s2
        even = slabs.at[2 * (s2 % 2)]
        odd = slabs.at[2 * (s2 % 2) + 1]
        nxte = slabs.at[2 * ((s2 + 1) % 2)]
        nxto = slabs.at[2 * ((s2 + 1) % 2) + 1]
        fire(se + 2, nxte)
        fire(se + 3, nxto)
        drain(se, even)
        oc1 = lax.cond(se < hi,
                       lambda: process(se - lo, even, ocount),
                       lambda: ocount)
        drain(se + 1, odd)
        oc2 = lax.cond(se + 1 < hi,
                       lambda: process(se + 1 - lo, odd, oc1),
                       lambda: oc1)
        return oc2

    ocount = lax.fori_loop(0, (spw + 1) // 2, step, jnp.int32(0))
    @pl.when(ocount >= 1)
    def _():
        pltpu.make_async_copy(ostage.at[0], out.at[pl.ds(0, 32)],
                              osem).wait()

    @pl.when(ocount >= 2)
    def _():
        pltpu.make_async_copy(ostage.at[1], out.at[pl.ds(0, 32)],
                              osem).wait()


def _sc_gather_body(uidx, iidx, uT, iT, uout, iout,
                    idx_v, hits, sub, slabs, ostage, bst, ssem, osem):
    wid = lax.axis_index("s") * NUM_CORES + lax.axis_index("c")
    pltpu.sync_copy(uidx, idx_v)
    _sweep_table(uT, uout, idx_v, hits, sub, slabs, ostage, bst,
                 ssem, osem, wid, U_SLABS, U_SPW)
    pltpu.sync_copy(iidx, idx_v)
    _sweep_table(iT, iout, idx_v, hits, sub, slabs, ostage, bst,
                 ssem, osem, wid, I_SLABS, I_SPW)


def _sc_gather(uidx, iidx, uT, iT):
    mesh = plsc.VectorSubcoreMesh(
        core_axis_name="c", subcore_axis_name="s",
        num_cores=NUM_CORES, num_subcores=NUM_SUBCORES)
    f = pl.kernel(
        _sc_gather_body,
        out_type=[
            jax.ShapeDtypeStruct((OUT_ROWS, 128), jnp.float32),
            jax.ShapeDtypeStruct((OUT_ROWS, 128), jnp.float32),
        ],
        mesh=mesh,
        compiler_params=pltpu.CompilerParams(needs_layout_passes=False),
        scratch_types=[
            pltpu.VMEM((BATCH,), jnp.int32),
            pltpu.VMEM((BATCH + 16,), jnp.int32),
            pltpu.VMEM((BATCH + 16,), jnp.int32),
            pltpu.VMEM((4, EMBED_DIM, SLAB), jnp.float32),
            pltpu.VMEM((2, 32, 128), jnp.float32),
            pltpu.VMEM((2, 32), jnp.int32),
            pltpu.SemaphoreType.DMA,
            pltpu.SemaphoreType.DMA,
        ],
    )
    return f(uidx, iidx, uT, iT)


def _mlp_body(uv_ref, iv_ref, uid_ref, iid_ref, utail_ref, itail_ref,
              aid_ref, gid_ref, aemb_ref, gemb_ref,
              w1_ref, b1_ref, w2_ref, b2_ref, w3_ref, b3_ref,
              wo_ref, bo_ref, out_ref):
    f32 = jnp.float32

    def dgt(x, w):
        return lax.dot_general(x, w, (((1,), (1,)), ((), ())),
                               preferred_element_type=f32)

    def with_tail(rows, ids, base, n, tail_ref):
        oh = (ids - base == lax.broadcasted_iota(jnp.int32, (1, n), 1))
        tv = jnp.dot(oh.astype(f32), tail_ref[...],
                     preferred_element_type=f32)
        return jnp.where(ids >= base, tv, rows[:, 0:EMBED_DIM])

    uv = with_tail(uv_ref[...], uid_ref[...], U_TAIL, 64, utail_ref)
    iv = with_tail(iv_ref[...], iid_ref[...], I_TAIL, 160, itail_ref)
    aid = aid_ref[...]
    gid = gid_ref[...]
    a_oh = (aid == lax.broadcasted_iota(jnp.int32, (1, 10), 1)).astype(f32)
    g_oh = (gid == lax.broadcasted_iota(jnp.int32, (1, 2), 1)).astype(f32)
    av = jnp.dot(a_oh, aemb_ref[...], preferred_element_type=f32)
    gv = jnp.dot(g_oh, gemb_ref[...], preferred_element_type=f32)
    w1 = w1_ref[...]
    h = (dgt(uv, w1[:, 0:32]) + dgt(iv, w1[:, 32:64])
         + dgt(av, w1[:, 64:96]) + dgt(gv, w1[:, 96:128]) + b1_ref[...])
    x = jnp.maximum(h, 0.0)
    x = jnp.maximum(dgt(x, w2_ref[...]) + b2_ref[...], 0.0)
    x = jnp.maximum(dgt(x, w3_ref[...]) + b3_ref[...], 0.0)
    o = jnp.sum(x * wo_ref[...], axis=1, keepdims=True) + bo_ref[0, 0]
    out_ref[...] = 1.0 / (1.0 + jnp.exp(-o))


def _mlp(uv, iv, uid, iid, utail, itail, aid, gid, age_emb, gender_emb,
         W1, b1, W2, b2, W3, b3, Wo, bo, interpret=False):
    nblk = BATCH // BLK
    full = lambda shape: pl.BlockSpec(shape, lambda i: (0, 0))
    batch_blk = lambda w: pl.BlockSpec((BLK, w), lambda i: (i, 0))
    return pl.pallas_call(
        _mlp_body,
        grid=(nblk,),
        in_specs=[
            batch_blk(128),
            batch_blk(128),
            batch_blk(1),
            batch_blk(1),
            full((64, EMBED_DIM)),
            full((160, EMBED_DIM)),
            batch_blk(1),
            batch_blk(1),
            full((10, EMBED_DIM)),
            full((2, EMBED_DIM)),
            full((64, 128)),
            full((1, 64)),
            full((32, 64)),
            full((1, 32)),
            full((16, 32)),
            full((1, 16)),
            full((1, 16)),
            pl.BlockSpec(memory_space=pltpu.SMEM),
        ],
        out_specs=batch_blk(1),
        out_shape=jax.ShapeDtypeStruct((BATCH, 1), jnp.float32),
        interpret=interpret,
    )(uv, iv, uid, iid, utail, itail, aid, gid, age_emb, gender_emb,
      W1, b1, W2, b2, W3, b3, Wo, bo)


@jax.jit
def kernel(user_input, item_input, age_input, gender_input, user_emb,
           item_emb, age_emb, gender_emb, W1, b1, W2, b2, W3, b3, Wo, bo):
    uidx = user_input.astype(jnp.int32)
    iidx = item_input.astype(jnp.int32)
    uvp, ivp = _sc_gather(uidx, iidx, user_emb.T, item_emb.T)
    aid = age_input.astype(jnp.int32).reshape(BATCH, 1)
    gid = gender_input.astype(jnp.int32).reshape(BATCH, 1)
    return _mlp(uvp, ivp,
                uidx.reshape(BATCH, 1), iidx.reshape(BATCH, 1),
                user_emb[U_TAIL:], item_emb[I_TAIL:],
                aid, gid, age_emb, gender_emb,
                W1, b1.reshape(1, 64), W2, b2.reshape(1, 32),
                W3, b3.reshape(1, 16), Wo, bo.reshape(1, 1))


# speedup vs baseline: 1.3540x; 1.3540x over previous
import jax
import jax.numpy as jnp
from jax import lax
from jax.experimental import pallas as pl
from jax.experimental.pallas import tpu as pltpu
from jax.experimental.pallas import tpu_sc as plsc

BATCH = 16384
EMBED_DIM = 32
NUM_CORES = 2
NUM_SUBCORES = 16
NUM_WORKERS = NUM_CORES * NUM_SUBCORES
USER_COUNT = 1000000
ITEM_COUNT = 100000
SLAB = 1024
U_FULL = USER_COUNT // SLAB
I_FULL = ITEM_COUNT // SLAB
U_SLABS = U_FULL + 1
I_SLABS = I_FULL + 1
U_TAIL = U_FULL * SLAB + 512
I_TAIL = I_FULL * SLAB + 512
U_SPW = -(-U_SLABS // NUM_WORKERS)
I_SPW = -(-I_SLABS // NUM_WORKERS)
OUT_ROWS = BATCH + NUM_WORKERS
BLK = 2048
_I16 = lambda: lax.iota(jnp.int32, 16)


def _sweep_table(tab, out, idx_v, hits, sub, slabs, ostage, bst,
                 ssem, osem, wid, n_slabs, spw, n_full, tbl_end):
    lo = wid * spw
    hi = jnp.minimum(lo + spw, n_slabs)
    dump = BATCH + wid

    def scan(k, cnt):
        v = idx_v[pl.ds(k * 16, 16)]
        sg = lax.shift_right_logical(v, 10)
        m = (sg >= lo) & (sg < hi) & (v < tbl_end)
        packed = (((sg - lo) << 24) | ((v & (SLAB - 1)) << 14)
                  | (k * 16 + _I16()))
        pos = cnt + plsc.cumsum(m.astype(jnp.int32)) - 1
        plsc.store_scatter(hits, [pos], packed, mask=m)
        return cnt + plsc.all_reduce_population_count(m)[0]

    cnt = lax.fori_loop(0, BATCH // 16, scan, jnp.int32(0))
    plsc.store_scatter(hits, [cnt + _I16()],
                       jnp.full((16,), 63 << 24, jnp.int32))
    nch = lax.shift_right_logical(cnt + 15, 4)

    def process(s_local, buf, ocount):
        cadj = jnp.where(lo + s_local == n_full, 512, 0)

        def rescan(t, scnt):
            hv = hits[pl.ds(t * 16, 16)]
            m = lax.shift_right_logical(hv, 24) == s_local
            pos = scnt + plsc.cumsum(m.astype(jnp.int32)) - 1
            plsc.store_scatter(sub, [pos], hv, mask=m)
            return scnt + plsc.all_reduce_population_count(m)[0]

        scnt = lax.fori_loop(0, nch, rescan, jnp.int32(0))

        def extract(e, oc):
            slot = oc & 1
            og = ostage.at[slot]

            @pl.when(oc >= 2)
            def _():
                pltpu.make_async_copy(og, out.at[pl.ds(0, 32)], osem).wait()

            for g in range(2):
                hv = sub[pl.ds(e * 32 + g * 16, 16)]
                col = (lax.shift_right_logical(hv, 14) & (SLAB - 1)) + cadj
                valid = (e * 32 + g * 16 + _I16()) < scnt
                b = jnp.where(valid, hv & (BATCH - 1), dump)
                bst.at[slot][pl.ds(g * 16, 16)] = b
                for f in range(EMBED_DIM):
                    vals = plsc.load_gather(
                        buf, [jnp.full((16,), f, jnp.int32), col])
                    plsc.store_scatter(
                        og, [g * 16 + _I16(), jnp.full((16,), f, jnp.int32)],
                        vals)
            pltpu.async_copy(og, out.at[bst.at[slot]], osem)
            return oc + 1

        nech = lax.shift_right_logical(scnt + 31, 5)
        return lax.fori_loop(0, nech, extract, ocount)

    def fire(sg, buf):
        @pl.when(sg < hi)
        def _():
            off = pl.multiple_of(
                jnp.where(sg == n_full, sg * SLAB - 512, sg * SLAB), 512)
            pltpu.async_copy(tab.at[:, pl.ds(off, SLAB)], buf, ssem)

    def drain(sg, buf):
        @pl.when(sg < hi)
        def _():
            pltpu.make_async_copy(tab.at[:, pl.ds(0, SLAB)], buf, ssem).wait()

    fire(lo, slabs.at[0])
    fire(lo + 1, slabs.at[1])

    def step(s, ocount):
        sg = lo + s
        cur = slabs.at[s % 2]
        drain(sg, cur)
        oc = lax.cond(sg < hi,
                      lambda: process(s, cur, ocount),
                      lambda: ocount)
        fire(sg + 2, cur)
        return oc

    ocount = lax.fori_loop(0, spw, step, jnp.int32(0))
    @pl.when(ocount >= 1)
    def _():
        pltpu.make_async_copy(ostage.at[0], out.at[pl.ds(0, 32)],
                              osem).wait()

    @pl.when(ocount >= 2)
    def _():
        pltpu.make_async_copy(ostage.at[1], out.at[pl.ds(0, 32)],
                              osem).wait()


def _sc_gather_body(uidx, iidx, uT, iT, uout, iout,
                    idx_v, hits, sub, slabs, ostage, bst, ssem, osem):
    wid = lax.axis_index("s") * NUM_CORES + lax.axis_index("c")
    pltpu.sync_copy(uidx, idx_v)
    _sweep_table(uT, uout, idx_v, hits, sub, slabs, ostage, bst,
                 ssem, osem, wid, U_SLABS, U_SPW, U_FULL, U_TAIL)
    pltpu.sync_copy(iidx, idx_v)
    _sweep_table(iT, iout, idx_v, hits, sub, slabs, ostage, bst,
                 ssem, osem, wid, I_SLABS, I_SPW, I_FULL, I_TAIL)


def _sc_gather(uidx, iidx, uT, iT):
    mesh = plsc.VectorSubcoreMesh(
        core_axis_name="c", subcore_axis_name="s",
        num_cores=NUM_CORES, num_subcores=NUM_SUBCORES)
    f = pl.kernel(
        _sc_gather_body,
        out_type=[
            jax.ShapeDtypeStruct((OUT_ROWS, 128), jnp.float32),
            jax.ShapeDtypeStruct((OUT_ROWS, 128), jnp.float32),
        ],
        mesh=mesh,
        compiler_params=pltpu.CompilerParams(needs_layout_passes=False),
        scratch_types=[
            pltpu.VMEM((BATCH,), jnp.int32),
            pltpu.VMEM((BATCH + 16,), jnp.int32),
            pltpu.VMEM((BATCH + 16,), jnp.int32),
            pltpu.VMEM((2, EMBED_DIM, SLAB), jnp.float32),
            pltpu.VMEM((2, 32, 128), jnp.float32),
            pltpu.VMEM((2, 32), jnp.int32),
            pltpu.SemaphoreType.DMA,
            pltpu.SemaphoreType.DMA,
        ],
    )
    return f(uidx, iidx, uT, iT)


def _mlp_body(uv_ref, iv_ref, uid_ref, iid_ref, utail_ref, itail_ref,
              aid_ref, gid_ref, aemb_ref, gemb_ref,
              w1_ref, b1_ref, w2_ref, b2_ref, w3_ref, b3_ref,
              wo_ref, bo_ref, out_ref):
    f32 = jnp.float32

    def dgt(x, w):
        return lax.dot_general(x, w, (((1,), (1,)), ((), ())),
                               preferred_element_type=f32)

    def with_tail(rows, ids, base, n, tail_ref):
        oh = (ids - base == lax.broadcasted_iota(jnp.int32, (1, n), 1))
        tv = jnp.dot(oh.astype(f32), tail_ref[...],
                     preferred_element_type=f32)
        return jnp.where(ids >= base, tv, rows[:, 0:EMBED_DIM])

    uv = with_tail(uv_ref[...], uid_ref[...], U_TAIL, 64, utail_ref)
    iv = with_tail(iv_ref[...], iid_ref[...], I_TAIL, 160, itail_ref)
    aid = aid_ref[...]
    gid = gid_ref[...]
    a_oh = (aid == lax.broadcasted_iota(jnp.int32, (1, 10), 1)).astype(f32)
    g_oh = (gid == lax.broadcasted_iota(jnp.int32, (1, 2), 1)).astype(f32)
    av = jnp.dot(a_oh, aemb_ref[...], preferred_element_type=f32)
    gv = jnp.dot(g_oh, gemb_ref[...], preferred_element_type=f32)
    w1 = w1_ref[...]
    h = (dgt(uv, w1[:, 0:32]) + dgt(iv, w1[:, 32:64])
         + dgt(av, w1[:, 64:96]) + dgt(gv, w1[:, 96:128]) + b1_ref[...])
    x = jnp.maximum(h, 0.0)
    x = jnp.maximum(dgt(x, w2_ref[...]) + b2_ref[...], 0.0)
    x = jnp.maximum(dgt(x, w3_ref[...]) + b3_ref[...], 0.0)
    o = jnp.sum(x * wo_ref[...], axis=1, keepdims=True) + bo_ref[0, 0]
    out_ref[...] = 1.0 / (1.0 + jnp.exp(-o))


def _mlp(uv, iv, uid, iid, utail, itail, aid, gid, age_emb, gender_emb,
         W1, b1, W2, b2, W3, b3, Wo, bo, interpret=False):
    nblk = BATCH // BLK
    full = lambda shape: pl.BlockSpec(shape, lambda i: (0, 0))
    batch_blk = lambda w: pl.BlockSpec((BLK, w), lambda i: (i, 0))
    return pl.pallas_call(
        _mlp_body,
        grid=(nblk,),
        in_specs=[
            batch_blk(128),
            batch_blk(128),
            batch_blk(1),
            batch_blk(1),
            full((64, EMBED_DIM)),
            full((160, EMBED_DIM)),
            batch_blk(1),
            batch_blk(1),
            full((10, EMBED_DIM)),
            full((2, EMBED_DIM)),
            full((64, 128)),
            full((1, 64)),
            full((32, 64)),
            full((1, 32)),
            full((16, 32)),
            full((1, 16)),
            full((1, 16)),
            pl.BlockSpec(memory_space=pltpu.SMEM),
        ],
        out_specs=batch_blk(1),
        out_shape=jax.ShapeDtypeStruct((BATCH, 1), jnp.float32),
        interpret=interpret,
    )(uv, iv, uid, iid, utail, itail, aid, gid, age_emb, gender_emb,
      W1, b1, W2, b2, W3, b3, Wo, bo)


@jax.jit
def kernel(user_input, item_input, age_input, gender_input, user_emb,
           item_emb, age_emb, gender_emb, W1, b1, W2, b2, W3, b3, Wo, bo):
    uidx = user_input.astype(jnp.int32)
    iidx = item_input.astype(jnp.int32)
    uvp, ivp = _sc_gather(uidx, iidx, user_emb.T, item_emb.T)
    aid = age_input.astype(jnp.int32).reshape(BATCH, 1)
    gid = gender_input.astype(jnp.int32).reshape(BATCH, 1)
    return _mlp(uvp, ivp,
                uidx.reshape(BATCH, 1), iidx.reshape(BATCH, 1),
                user_emb[U_TAIL:], item_emb[I_TAIL:],
                aid, gid, age_emb, gender_emb,
                W1, b1.reshape(1, 64), W2, b2.reshape(1, 32),
                W3, b3.reshape(1, 16), Wo, bo.reshape(1, 1))


# speedup vs baseline: 1.3669x; 1.0095x over previous
import jax
import jax.numpy as jnp
from jax import lax
from jax.experimental import pallas as pl
from jax.experimental.pallas import tpu as pltpu
from jax.experimental.pallas import tpu_sc as plsc

BATCH = 16384
EMBED_DIM = 32
NUM_CORES = 2
NUM_SUBCORES = 16
NUM_WORKERS = NUM_CORES * NUM_SUBCORES
USER_COUNT = 1000000
ITEM_COUNT = 100000
SLAB = 1024
U_FULL = USER_COUNT // SLAB
I_FULL = ITEM_COUNT // SLAB
U_SLABS = U_FULL + 1
I_SLABS = I_FULL + 1
U_TAIL = U_FULL * SLAB + 512
I_TAIL = I_FULL * SLAB + 512
U_SPW = -(-U_SLABS // NUM_WORKERS)
I_SPW = -(-I_SLABS // NUM_WORKERS)
OUT_ROWS = BATCH + NUM_WORKERS
BLK = 2048
_I16 = lambda: lax.iota(jnp.int32, 16)


def _sweep_table(tab, out, idx_v, hits, sub, slabs, ostage, bst,
                 ssem, osem, wid, n_slabs, spw, n_full, tbl_end):
    lo = wid * spw
    hi = jnp.minimum(lo + spw, n_slabs)
    dump = BATCH + wid

    def fire(sg, buf):
        @pl.when(sg < hi)
        def _():
            off = pl.multiple_of(
                jnp.where(sg == n_full, sg * SLAB - 512, sg * SLAB), 512)
            pltpu.async_copy(tab.at[:, pl.ds(off, SLAB)], buf, ssem)

    def drain(sg, buf):
        @pl.when(sg < hi)
        def _():
            pltpu.make_async_copy(tab.at[:, pl.ds(0, SLAB)], buf, ssem).wait()

    fire(lo, slabs.at[0])
    fire(lo + 1, slabs.at[1])

    def scan(k, cnt):
        v = idx_v[pl.ds(k * 16, 16)]
        sg = lax.shift_right_logical(v, 10)
        m = (sg >= lo) & (sg < hi) & (v < tbl_end)
        packed = (((sg - lo) << 24) | ((v & (SLAB - 1)) << 14)
                  | (k * 16 + _I16()))
        pos = cnt + plsc.cumsum(m.astype(jnp.int32)) - 1
        plsc.store_scatter(hits, [pos], packed, mask=m)
        return cnt + plsc.all_reduce_population_count(m)[0]

    cnt = lax.fori_loop(0, BATCH // 16, scan, jnp.int32(0))
    plsc.store_scatter(hits, [cnt + _I16()],
                       jnp.full((16,), 63 << 24, jnp.int32))
    nch = lax.shift_right_logical(cnt + 15, 4)

    def process(s_local, buf, ocount):
        cadj = jnp.where(lo + s_local == n_full, 512, 0)

        def rescan(t, scnt):
            hv = hits[pl.ds(t * 16, 16)]
            m = lax.shift_right_logical(hv, 24) == s_local
            pos = scnt + plsc.cumsum(m.astype(jnp.int32)) - 1
            plsc.store_scatter(sub, [pos], hv, mask=m)
            return scnt + plsc.all_reduce_population_count(m)[0]

        scnt = lax.fori_loop(0, nch, rescan, jnp.int32(0))

        def extract(e, oc):
            slot = oc & 1
            og = ostage.at[slot]

            @pl.when(oc >= 2)
            def _():
                pltpu.make_async_copy(og, out.at[pl.ds(0, 32)], osem).wait()

            for g in range(2):
                hv = sub[pl.ds(e * 32 + g * 16, 16)]
                col = (lax.shift_right_logical(hv, 14) & (SLAB - 1)) + cadj
                valid = (e * 32 + g * 16 + _I16()) < scnt
                b = jnp.where(valid, hv & (BATCH - 1), dump)
                bst.at[slot][pl.ds(g * 16, 16)] = b
                for f in range(EMBED_DIM):
                    vals = plsc.load_gather(
                        buf, [jnp.full((16,), f, jnp.int32), col])
                    plsc.store_scatter(
                        og, [g * 16 + _I16(), jnp.full((16,), f, jnp.int32)],
                        vals)
            pltpu.async_copy(og, out.at[bst.at[slot]], osem)
            return oc + 1

        nech = lax.shift_right_logical(scnt + 31, 5)
        return lax.fori_loop(0, nech, extract, ocount)

    def step(s, ocount):
        sg = lo + s
        cur = slabs.at[s % 2]
        drain(sg, cur)
        oc = lax.cond(sg < hi,
                      lambda: process(s, cur, ocount),
                      lambda: ocount)
        fire(sg + 2, cur)
        return oc

    ocount = lax.fori_loop(0, spw, step, jnp.int32(0))
    @pl.when(ocount >= 1)
    def _():
        pltpu.make_async_copy(ostage.at[0], out.at[pl.ds(0, 32)],
                              osem).wait()

    @pl.when(ocount >= 2)
    def _():
        pltpu.make_async_copy(ostage.at[1], out.at[pl.ds(0, 32)],
                              osem).wait()


def _sc_gather_body(uidx, iidx, uT, iT, uout, iout,
                    idx_v, hits, sub, slabs, ostage, bst, ssem, osem):
    wid = lax.axis_index("s") * NUM_CORES + lax.axis_index("c")
    pltpu.sync_copy(uidx, idx_v)
    _sweep_table(uT, uout, idx_v, hits, sub, slabs, ostage, bst,
                 ssem, osem, wid, U_SLABS, U_SPW, U_FULL, U_TAIL)
    pltpu.sync_copy(iidx, idx_v)
    _sweep_table(iT, iout, idx_v, hits, sub, slabs, ostage, bst,
                 ssem, osem, wid, I_SLABS, I_SPW, I_FULL, I_TAIL)


def _sc_gather(uidx, iidx, uT, iT):
    mesh = plsc.VectorSubcoreMesh(
        core_axis_name="c", subcore_axis_name="s",
        num_cores=NUM_CORES, num_subcores=NUM_SUBCORES)
    f = pl.kernel(
        _sc_gather_body,
        out_type=[
            jax.ShapeDtypeStruct((OUT_ROWS, 128), jnp.float32),
            jax.ShapeDtypeStruct((OUT_ROWS, 128), jnp.float32),
        ],
        mesh=mesh,
        compiler_params=pltpu.CompilerParams(needs_layout_passes=False),
        scratch_types=[
            pltpu.VMEM((BATCH,), jnp.int32),
            pltpu.VMEM((BATCH + 16,), jnp.int32),
            pltpu.VMEM((BATCH + 16,), jnp.int32),
            pltpu.VMEM((2, EMBED_DIM, SLAB), jnp.float32),
            pltpu.VMEM((2, 32, 128), jnp.float32),
            pltpu.VMEM((2, 32), jnp.int32),
            pltpu.SemaphoreType.DMA,
            pltpu.SemaphoreType.DMA,
        ],
    )
    return f(uidx, iidx, uT, iT)


def _mlp_body(uv_ref, iv_ref, uid_ref, iid_ref, utail_ref, itail_ref,
              aid_ref, gid_ref, aemb_ref, gemb_ref,
              w1_ref, b1_ref, w2_ref, b2_ref, w3_ref, b3_ref,
              wo_ref, bo_ref, out_ref):
    f32 = jnp.float32

    def dgt(x, w):
        return lax.dot_general(x, w, (((1,), (1,)), ((), ())),
                               preferred_element_type=f32)

    def with_tail(rows, ids, base, n, tail_ref):
        oh = (ids - base == lax.broadcasted_iota(jnp.int32, (1, n), 1))
        tv = jnp.dot(oh.astype(f32), tail_ref[...],
                     preferred_element_type=f32)
        return jnp.where(ids >= base, tv, rows[:, 0:EMBED_DIM])

    uv = with_tail(uv_ref[...], uid_ref[...], U_TAIL, 64, utail_ref)
    iv = with_tail(iv_ref[...], iid_ref[...], I_TAIL, 160, itail_ref)
    aid = aid_ref[...]
    gid = gid_ref[...]
    a_oh = (aid == lax.broadcasted_iota(jnp.int32, (1, 10), 1)).astype(f32)
    g_oh = (gid == lax.broadcasted_iota(jnp.int32, (1, 2), 1)).astype(f32)
    av = jnp.dot(a_oh, aemb_ref[...], preferred_element_type=f32)
    gv = jnp.dot(g_oh, gemb_ref[...], preferred_element_type=f32)
    w1 = w1_ref[...]
    h = (dgt(uv, w1[:, 0:32]) + dgt(iv, w1[:, 32:64])
         + dgt(av, w1[:, 64:96]) + dgt(gv, w1[:, 96:128]) + b1_ref[...])
    x = jnp.maximum(h, 0.0)
    x = jnp.maximum(dgt(x, w2_ref[...]) + b2_ref[...], 0.0)
    x = jnp.maximum(dgt(x, w3_ref[...]) + b3_ref[...], 0.0)
    o = jnp.sum(x * wo_ref[...], axis=1, keepdims=True) + bo_ref[0, 0]
    out_ref[...] = 1.0 / (1.0 + jnp.exp(-o))


def _mlp(uv, iv, uid, iid, utail, itail, aid, gid, age_emb, gender_emb,
         W1, b1, W2, b2, W3, b3, Wo, bo, interpret=False):
    nblk = BATCH // BLK
    full = lambda shape: pl.BlockSpec(shape, lambda i: (0, 0))
    batch_blk = lambda w: pl.BlockSpec((BLK, w), lambda i: (i, 0))
    return pl.pallas_call(
        _mlp_body,
        grid=(nblk,),
        in_specs=[
            batch_blk(128),
            batch_blk(128),
            batch_blk(1),
            batch_blk(1),
            full((64, EMBED_DIM)),
            full((160, EMBED_DIM)),
            batch_blk(1),
            batch_blk(1),
            full((10, EMBED_DIM)),
            full((2, EMBED_DIM)),
            full((64, 128)),
            full((1, 64)),
            full((32, 64)),
            full((1, 32)),
            full((16, 32)),
            full((1, 16)),
            full((1, 16)),
            pl.BlockSpec(memory_space=pltpu.SMEM),
        ],
        out_specs=batch_blk(1),
        out_shape=jax.ShapeDtypeStruct((BATCH, 1), jnp.float32),
        interpret=interpret,
    )(uv, iv, uid, iid, utail, itail, aid, gid, age_emb, gender_emb,
      W1, b1, W2, b2, W3, b3, Wo, bo)


@jax.jit
def kernel(user_input, item_input, age_input, gender_input, user_emb,
           item_emb, age_emb, gender_emb, W1, b1, W2, b2, W3, b3, Wo, bo):
    uidx = user_input.astype(jnp.int32)
    iidx = item_input.astype(jnp.int32)
    uvp, ivp = _sc_gather(uidx, iidx, user_emb.T, item_emb.T)
    aid = age_input.astype(jnp.int32).reshape(BATCH, 1)
    gid = gender_input.astype(jnp.int32).reshape(BATCH, 1)
    return _mlp(uvp, ivp,
                uidx.reshape(BATCH, 1), iidx.reshape(BATCH, 1),
                user_emb[U_TAIL:], item_emb[I_TAIL:],
                aid, gid, age_emb, gender_emb,
                W1, b1.reshape(1, 64), W2, b2.reshape(1, 32),
                W3, b3.reshape(1, 16), Wo, bo.reshape(1, 1))


# speedup vs baseline: 1.3832x; 1.0119x over previous
import jax
import jax.numpy as jnp
from jax import lax
from jax.experimental import pallas as pl
from jax.experimental.pallas import tpu as pltpu
from jax.experimental.pallas import tpu_sc as plsc

BATCH = 16384
EMBED_DIM = 32
NUM_CORES = 2
NUM_SUBCORES = 16
NUM_WORKERS = NUM_CORES * NUM_SUBCORES
USER_COUNT = 1000000
ITEM_COUNT = 100000
SLAB = 1024
U_FULL = USER_COUNT // SLAB
I_FULL = ITEM_COUNT // SLAB
U_SLABS = U_FULL + 1
I_SLABS = I_FULL + 1
U_TAIL = (USER_COUNT // 128) * 128
I_TAIL = (ITEM_COUNT // 128) * 128
U_TCT = USER_COUNT - U_TAIL
I_TCT = ITEM_COUNT - I_TAIL
U_SPW = -(-U_SLABS // NUM_WORKERS)
I_SPW = -(-I_SLABS // NUM_WORKERS)
OUT_ROWS = BATCH + NUM_WORKERS
BLK = 2048
_I16 = lambda: lax.iota(jnp.int32, 16)


def _sweep_table(tab, out, idx_v, hits, sub, slabs, ostage, bst,
                 ssem, osem, wid, n_slabs, spw, n_full, tbl_end):
    lo = wid * spw
    hi = jnp.minimum(lo + spw, n_slabs)
    dump = BATCH + wid

    def fire(sg, buf):
        @pl.when(sg < hi)
        def _():
            off = pl.multiple_of(
                jnp.where(sg == n_full, tbl_end - SLAB, sg * SLAB), 128)
            pltpu.async_copy(tab.at[:, pl.ds(off, SLAB)], buf, ssem)

    def drain(sg, buf):
        @pl.when(sg < hi)
        def _():
            pltpu.make_async_copy(tab.at[:, pl.ds(0, SLAB)], buf, ssem).wait()

    fire(lo, slabs.at[0])
    fire(lo + 1, slabs.at[1])

    def scan(k, cnt):
        v = idx_v[pl.ds(k * 16, 16)]
        sg = lax.shift_right_logical(v, 10)
        m = (sg >= lo) & (sg < hi) & (v < tbl_end)
        packed = (((sg - lo) << 24) | ((v & (SLAB - 1)) << 14)
                  | (k * 16 + _I16()))
        pos = cnt + plsc.cumsum(m.astype(jnp.int32)) - 1
        plsc.store_scatter(hits, [pos], packed, mask=m)
        return cnt + plsc.all_reduce_population_count(m)[0]

    cnt = lax.fori_loop(0, BATCH // 16, scan, jnp.int32(0))
    plsc.store_scatter(hits, [cnt + _I16()],
                       jnp.full((16,), 63 << 24, jnp.int32))
    nch = lax.shift_right_logical(cnt + 15, 4)

    def process(s_local, buf, ocount):
        cadj = jnp.where(lo + s_local == n_full,
                         (n_full + 1) * SLAB - tbl_end, 0)

        def rescan(t, scnt):
            hv = hits[pl.ds(t * 16, 16)]
            m = lax.shift_right_logical(hv, 24) == s_local
            pos = scnt + plsc.cumsum(m.astype(jnp.int32)) - 1
            plsc.store_scatter(sub, [pos], hv, mask=m)
            return scnt + plsc.all_reduce_population_count(m)[0]

        scnt = lax.fori_loop(0, nch, rescan, jnp.int32(0))

        def extract(e, oc):
            slot = oc & 1
            og = ostage.at[slot]

            @pl.when(oc >= 2)
            def _():
                pltpu.make_async_copy(og, out.at[pl.ds(0, 32)], osem).wait()

            for g in range(2):
                hv = sub[pl.ds(e * 32 + g * 16, 16)]
                col = (lax.shift_right_logical(hv, 14) & (SLAB - 1)) + cadj
                valid = (e * 32 + g * 16 + _I16()) < scnt
                b = jnp.where(valid, hv & (BATCH - 1), dump)
                bst.at[slot][pl.ds(g * 16, 16)] = b
                for f in range(EMBED_DIM):
                    vals = plsc.load_gather(
                        buf, [jnp.full((16,), f, jnp.int32), col])
                    plsc.store_scatter(
                        og, [g * 16 + _I16(), jnp.full((16,), f, jnp.int32)],
                        vals)
            pltpu.async_copy(og, out.at[bst.at[slot]], osem)
            return oc + 1

        nech = lax.shift_right_logical(scnt + 31, 5)
        return lax.fori_loop(0, nech, extract, ocount)

    def step(s, ocount):
        sg = lo + s
        cur = slabs.at[s % 2]
        drain(sg, cur)
        oc = lax.cond(sg < hi,
                      lambda: process(s, cur, ocount),
                      lambda: ocount)
        fire(sg + 2, cur)
        return oc

    ocount = lax.fori_loop(0, spw, step, jnp.int32(0))
    @pl.when(ocount >= 1)
    def _():
        pltpu.make_async_copy(ostage.at[0], out.at[pl.ds(0, 32)],
                              osem).wait()

    @pl.when(ocount >= 2)
    def _():
        pltpu.make_async_copy(ostage.at[1], out.at[pl.ds(0, 32)],
                              osem).wait()


def _sc_gather_body(uidx, iidx, uT, iT, uout, iout,
                    idx_v, hits, sub, slabs, ostage, bst, ssem, osem):
    wid = lax.axis_index("s") * NUM_CORES + lax.axis_index("c")
    pltpu.sync_copy(uidx, idx_v)
    _sweep_table(uT, uout, idx_v, hits, sub, slabs, ostage, bst,
                 ssem, osem, wid, U_SLABS, U_SPW, U_FULL, U_TAIL)
    pltpu.sync_copy(iidx, idx_v)
    _sweep_table(iT, iout, idx_v, hits, sub, slabs, ostage, bst,
                 ssem, osem, wid, I_SLABS, I_SPW, I_FULL, I_TAIL)


def _sc_gather(uidx, iidx, uT, iT):
    mesh = plsc.VectorSubcoreMesh(
        core_axis_name="c", subcore_axis_name="s",
        num_cores=NUM_CORES, num_subcores=NUM_SUBCORES)
    f = pl.kernel(
        _sc_gather_body,
        out_type=[
            jax.ShapeDtypeStruct((OUT_ROWS, 128), jnp.float32),
            jax.ShapeDtypeStruct((OUT_ROWS, 128), jnp.float32),
        ],
        mesh=mesh,
        compiler_params=pltpu.CompilerParams(needs_layout_passes=False),
        scratch_types=[
            pltpu.VMEM((BATCH,), jnp.int32),
            pltpu.VMEM((BATCH + 16,), jnp.int32),
            pltpu.VMEM((BATCH + 16,), jnp.int32),
            pltpu.VMEM((2, EMBED_DIM, SLAB), jnp.float32),
            pltpu.VMEM((2, 32, 128), jnp.float32),
            pltpu.VMEM((2, 32), jnp.int32),
            pltpu.SemaphoreType.DMA,
            pltpu.SemaphoreType.DMA,
        ],
    )
    return f(uidx, iidx, uT, iT)


def _mlp_body(uv_ref, iv_ref, uid_ref, iid_ref, utail_ref, itail_ref,
              aid_ref, gid_ref, aemb_ref, gemb_ref,
              w1_ref, b1_ref, w2_ref, b2_ref, w3_ref, b3_ref,
              wo_ref, bo_ref, out_ref):
    f32 = jnp.float32

    def dgt(x, w):
        return lax.dot_general(x, w, (((1,), (1,)), ((), ())),
                               preferred_element_type=f32)

    def with_tail(rows, ids, base, n, tail_ref):
        oh = (ids - base == lax.broadcasted_iota(jnp.int32, (1, n), 1))
        tv = jnp.dot(oh.astype(f32), tail_ref[...],
                     preferred_element_type=f32)
        return jnp.where(ids >= base, tv, rows[:, 0:EMBED_DIM])

    uv = with_tail(uv_ref[...], uid_ref[...], U_TAIL, U_TCT, utail_ref)
    iv = with_tail(iv_ref[...], iid_ref[...], I_TAIL, I_TCT, itail_ref)
    aid = aid_ref[...]
    gid = gid_ref[...]
    a_oh = (aid == lax.broadcasted_iota(jnp.int32, (1, 10), 1)).astype(f32)
    g_oh = (gid == lax.broadcasted_iota(jnp.int32, (1, 2), 1)).astype(f32)
    w1 = w1_ref[...]
    a2 = dgt(aemb_ref[...], w1[:, 64:96])
    g2 = dgt(gemb_ref[...], w1[:, 96:128])
    h = (dgt(uv, w1[:, 0:32]) + dgt(iv, w1[:, 32:64])
         + jnp.dot(a_oh, a2, preferred_element_type=f32)
         + jnp.dot(g_oh, g2, preferred_element_type=f32) + b1_ref[...])
    x = jnp.maximum(h, 0.0)
    x = jnp.maximum(dgt(x, w2_ref[...]) + b2_ref[...], 0.0)
    x = jnp.maximum(dgt(x, w3_ref[...]) + b3_ref[...], 0.0)
    o = jnp.sum(x * wo_ref[...], axis=1, keepdims=True) + bo_ref[0, 0]
    out_ref[...] = 1.0 / (1.0 + jnp.exp(-o))


def _mlp(uv, iv, uid, iid, utail, itail, aid, gid, age_emb, gender_emb,
         W1, b1, W2, b2, W3, b3, Wo, bo, interpret=False):
    nblk = BATCH // BLK
    full = lambda shape: pl.BlockSpec(shape, lambda i: (0, 0))
    batch_blk = lambda w: pl.BlockSpec((BLK, w), lambda i: (i, 0))
    return pl.pallas_call(
        _mlp_body,
        grid=(nblk,),
        in_specs=[
            batch_blk(128),
            batch_blk(128),
            batch_blk(1),
            batch_blk(1),
            full((U_TCT, EMBED_DIM)),
            full((I_TCT, EMBED_DIM)),
            batch_blk(1),
            batch_blk(1),
            full((10, EMBED_DIM)),
            full((2, EMBED_DIM)),
            full((64, 128)),
            full((1, 64)),
            full((32, 64)),
            full((1, 32)),
            full((16, 32)),
            full((1, 16)),
            full((1, 16)),
            pl.BlockSpec(memory_space=pltpu.SMEM),
        ],
        out_specs=batch_blk(1),
        out_shape=jax.ShapeDtypeStruct((BATCH, 1), jnp.float32),
        interpret=interpret,
    )(uv, iv, uid, iid, utail, itail, aid, gid, age_emb, gender_emb,
      W1, b1, W2, b2, W3, b3, Wo, bo)


@jax.jit
def kernel(user_input, item_input, age_input, gender_input, user_emb,
           item_emb, age_emb, gender_emb, W1, b1, W2, b2, W3, b3, Wo, bo):
    uidx = user_input.astype(jnp.int32)
    iidx = item_input.astype(jnp.int32)
    uvp, ivp = _sc_gather(uidx, iidx, user_emb.T, item_emb.T)
    aid = age_input.astype(jnp.int32).reshape(BATCH, 1)
    gid = gender_input.astype(jnp.int32).reshape(BATCH, 1)
    return _mlp(uvp, ivp,
                uidx.reshape(BATCH, 1), iidx.reshape(BATCH, 1),
                user_emb[U_TAIL:], item_emb[I_TAIL:],
                aid, gid, age_emb, gender_emb,
                W1, b1.reshape(1, 64), W2, b2.reshape(1, 32),
                W3, b3.reshape(1, 16), Wo, bo.reshape(1, 1))
